# Initial kernel scaffold; baseline (speedup 1.0000x reference)
import jax
import jax.numpy as jnp
from jax.experimental import pallas as pl


def kernel(x, edge_index, edge_weight, Wrel0, brel0, Wroot0, Wrel1, brel1, Wroot1, Wrel2, brel2, Wroot2, Wrel3, brel3, Wroot3, Wrel4, brel4, Wroot4):
    raise NotImplementedError("write your pallas kernel here")


# speedup vs baseline: 6.6719x; 6.6719x over previous
import jax
import jax.numpy as jnp
from jax import lax
from jax.experimental import pallas as pl
from jax.experimental.pallas import tpu as pltpu
from jax.experimental.pallas import tpu_sc as plsc

N_NODES = 50000
NP = 51200
NROWS = NP // 128
NE = 819200
NEROWS = NE // 128
HALF = 32
BATCH = 128
CHUNK = 40

_mesh = plsc.VectorSubcoreMesh(core_axis_name="c", subcore_axis_name="s")


def _agg64_body(y_ref, src_ref, dst_ref, ew_ref, out_ref,
                sidx, didx, ewb, ybuf, mbuf, acc, gsem, ssem):
    c = lax.axis_index("c")
    s = lax.axis_index("s")
    zero16 = jnp.zeros((16,), jnp.float32)

    for j in range(BATCH):
        mbuf[j, pl.ds(0, 16)] = zero16
        mbuf[j, pl.ds(16, 16)] = zero16

    def zcp(i, _):
        pltpu.sync_copy(mbuf, acc.at[pl.ds(s * (NP // 16) + i * BATCH, BATCH)])
        return 0

    lax.fori_loop(0, (NP // 16) // BATCH, zcp, 0)
    plsc.subcore_barrier()

    rows_per_sub = NEROWS // 16

    def chunk_body(k, _):
        ebase = s * rows_per_sub + k * CHUNK
        pltpu.sync_copy(src_ref.at[pl.ds(c * NEROWS + ebase, CHUNK)], sidx)
        pltpu.sync_copy(dst_ref.at[pl.ds(ebase, CHUNK)], didx)
        pltpu.sync_copy(ew_ref.at[pl.ds(ebase, CHUNK)], ewb)

        def batch_body(b, _):
            pltpu.async_copy(y_ref.at[sidx.at[b]], ybuf, gsem).wait()
            for g in range(8):
                ew16 = ewb[b, pl.ds(g * 16, 16)]
                for j in range(16):
                    e = g * 16 + j
                    es = ew16[jnp.full((16,), j, jnp.int32)]
                    mbuf[e, pl.ds(0, 16)] = ybuf[e, pl.ds(0, 16)] * es
                    mbuf[e, pl.ds(16, 16)] = ybuf[e, pl.ds(16, 16)] * es
            pltpu.async_copy(mbuf, acc.at[didx.at[b]], ssem, add=True).wait()
            return 0

        lax.fori_loop(0, CHUNK, batch_body, 0)
        return 0

    lax.fori_loop(0, rows_per_sub // CHUNK, chunk_body, 0)
    plsc.subcore_barrier()
    pltpu.sync_copy(acc.at[pl.ds(s * (NP // 16), NP // 16)],
                    out_ref.at[pl.ds(c * NP + s * (NP // 16), NP // 16)])


@jax.jit
def _sc_agg64(yflat, srcoff, dst2d, ew2d):
    return pl.kernel(
        _agg64_body,
        out_type=jax.ShapeDtypeStruct((2 * NP, HALF), jnp.float32),
        mesh=_mesh,
        compiler_params=pltpu.CompilerParams(use_tc_tiling_on_sc=False),
        scratch_types=[
            pltpu.VMEM((CHUNK, BATCH), jnp.int32),
            pltpu.VMEM((CHUNK, BATCH), jnp.int32),
            pltpu.VMEM((CHUNK, BATCH), jnp.float32),
            pltpu.VMEM((BATCH, HALF), jnp.float32),
            pltpu.VMEM((BATCH, HALF), jnp.float32),
            pltpu.VMEM_SHARED((NP, HALF), jnp.float32),
            pltpu.SemaphoreType.DMA,
            pltpu.SemaphoreType.DMA,
        ],
    )(yflat, srcoff, dst2d, ew2d)


def _agg1_body(v_ref, src_ref, dst_ref, ew_ref, out_ref,
               sidx, didx, ewb, xbuf, mbuf, zbuf, acc, gsem, ssem):
    c = lax.axis_index("c")
    s = lax.axis_index("s")
    zero16 = jnp.zeros((16,), jnp.float32)

    def zb(i, _):
        zbuf[pl.ds(i * 16, 16)] = zero16
        return 0

    lax.fori_loop(0, (NP // 16) // 16, zb, 0)
    pltpu.sync_copy(zbuf, acc.at[pl.ds(s * (NP // 16), NP // 16)])
    plsc.subcore_barrier()

    rows_per_sub = (NEROWS // 2) // 16

    def chunk_body(k, _):
        ebase = c * (NEROWS // 2) + s * rows_per_sub + k * CHUNK
        pltpu.sync_copy(src_ref.at[pl.ds(ebase, CHUNK)], sidx)
        pltpu.sync_copy(dst_ref.at[pl.ds(ebase, CHUNK)], didx)
        pltpu.sync_copy(ew_ref.at[pl.ds(ebase, CHUNK)], ewb)

        def batch_body(b, _):
            pltpu.async_copy(v_ref.at[sidx.at[b]], xbuf, gsem).wait()
            for g in range(8):
                mbuf[pl.ds(g * 16, 16)] = (xbuf[pl.ds(g * 16, 16)]
                                           * ewb[b, pl.ds(g * 16, 16)])
            pltpu.async_copy(mbuf, acc.at[didx.at[b]], ssem, add=True).wait()
            return 0

        lax.fori_loop(0, CHUNK, batch_body, 0)
        return 0

    lax.fori_loop(0, rows_per_sub // CHUNK, chunk_body, 0)
    plsc.subcore_barrier()
    pltpu.sync_copy(acc.at[pl.ds(s * (NP // 16), NP // 16)],
                    out_ref.at[pl.ds(c * NP + s * (NP // 16), NP // 16)])


@jax.jit
def _sc_agg1(v, src2d, dst2d, ew2d):
    return pl.kernel(
        _agg1_body,
        out_type=jax.ShapeDtypeStruct((2 * NP,), jnp.float32),
        mesh=_mesh,
        compiler_params=pltpu.CompilerParams(use_tc_tiling_on_sc=False),
        scratch_types=[
            pltpu.VMEM((CHUNK, BATCH), jnp.int32),
            pltpu.VMEM((CHUNK, BATCH), jnp.int32),
            pltpu.VMEM((CHUNK, BATCH), jnp.float32),
            pltpu.VMEM((BATCH,), jnp.float32),
            pltpu.VMEM((BATCH,), jnp.float32),
            pltpu.VMEM((NP // 16,), jnp.float32),
            pltpu.VMEM_SHARED((NP,), jnp.float32),
            pltpu.SemaphoreType.DMA,
            pltpu.SemaphoreType.DMA,
        ],
    )(v, src2d, dst2d, ew2d)


BLK = 1024
GRID = NP // BLK


def _tc1_body(x_ref, agg0_ref, wr0_ref, br0_ref, wk0_ref, wr1_ref, wk1_ref,
              br1_ref, y_ref, r_ref):
    a = agg0_ref[0] + agg0_ref[1]
    x = x_ref[...]
    h = jnp.maximum(a * wr0_ref[...] + br0_ref[...] + x * wk0_ref[...], 0.0)
    y = jnp.dot(h, wr1_ref[...], preferred_element_type=jnp.float32)
    y_ref[0] = y[:, :HALF]
    y_ref[1] = y[:, HALF:]
    r_ref[...] = (jnp.dot(h, wk1_ref[...], preferred_element_type=jnp.float32)
                  + br1_ref[...])


@jax.jit
def _tc1(x2d, agg0, wr0, br0, wk0, wr1, wk1, br1):
    return pl.pallas_call(
        _tc1_body,
        grid=(GRID,),
        in_specs=[
            pl.BlockSpec((BLK, 1), lambda i: (i, 0)),
            pl.BlockSpec((2, BLK, 1), lambda i: (0, i, 0)),
            pl.BlockSpec((1, 64), lambda i: (0, 0)),
            pl.BlockSpec((1, 64), lambda i: (0, 0)),
            pl.BlockSpec((1, 64), lambda i: (0, 0)),
            pl.BlockSpec((64, 64), lambda i: (0, 0)),
            pl.BlockSpec((64, 64), lambda i: (0, 0)),
            pl.BlockSpec((1, 64), lambda i: (0, 0)),
        ],
        out_specs=[
            pl.BlockSpec((2, BLK, HALF), lambda i: (0, i, 0)),
            pl.BlockSpec((BLK, 64), lambda i: (i, 0)),
        ],
        out_shape=[
            jax.ShapeDtypeStruct((2, NP, HALF), jnp.float32),
            jax.ShapeDtypeStruct((NP, 64), jnp.float32),
        ],
    )(x2d, agg0, wr0, br0, wk0, wr1, wk1, br1)


def _tcmid_body(agg_ref, rin_ref, wr_ref, wk_ref, br_ref, y_ref, r_ref):
    h = jnp.maximum(
        jnp.concatenate([agg_ref[0], agg_ref[1]], axis=-1) + rin_ref[...], 0.0)
    y = jnp.dot(h, wr_ref[...], preferred_element_type=jnp.float32)
    y_ref[0] = y[:, :HALF]
    y_ref[1] = y[:, HALF:]
    r_ref[...] = (jnp.dot(h, wk_ref[...], preferred_element_type=jnp.float32)
                  + br_ref[...])


@jax.jit
def _tcmid(agg, rin, wr, wk, br):
    return pl.pallas_call(
        _tcmid_body,
        grid=(GRID,),
        in_specs=[
            pl.BlockSpec((2, BLK, HALF), lambda i: (0, i, 0)),
            pl.BlockSpec((BLK, 64), lambda i: (i, 0)),
            pl.BlockSpec((64, 64), lambda i: (0, 0)),
            pl.BlockSpec((64, 64), lambda i: (0, 0)),
            pl.BlockSpec((1, 64), lambda i: (0, 0)),
        ],
        out_specs=[
            pl.BlockSpec((2, BLK, HALF), lambda i: (0, i, 0)),
            pl.BlockSpec((BLK, 64), lambda i: (i, 0)),
        ],
        out_shape=[
            jax.ShapeDtypeStruct((2, NP, HALF), jnp.float32),
            jax.ShapeDtypeStruct((NP, 64), jnp.float32),
        ],
    )(agg, rin, wr, wk, br)


def _tc4_body(agg_ref, rin_ref, wr4_ref, wk4_ref, br4_ref, y4_ref, r4_ref):
    h = jnp.maximum(
        jnp.concatenate([agg_ref[0], agg_ref[1]], axis=-1) + rin_ref[...], 0.0)
    y4 = jnp.dot(h, wr4_ref[...], preferred_element_type=jnp.float32)
    r4 = (jnp.dot(h, wk4_ref[...], preferred_element_type=jnp.float32)
          + br4_ref[...])
    y4_ref[...] = y4
    r4_ref[...] = r4


@jax.jit
def _tc4(agg, rin, wr4, wk4, br4):
    return pl.pallas_call(
        _tc4_body,
        grid=(GRID,),
        in_specs=[
            pl.BlockSpec((2, BLK, HALF), lambda i: (0, i, 0)),
            pl.BlockSpec((BLK, 64), lambda i: (i, 0)),
            pl.BlockSpec((64, 1), lambda i: (0, 0)),
            pl.BlockSpec((64, 1), lambda i: (0, 0)),
            pl.BlockSpec((1, 1), lambda i: (0, 0)),
        ],
        out_specs=[
            pl.BlockSpec((BLK, 1), lambda i: (i, 0)),
            pl.BlockSpec((BLK, 1), lambda i: (i, 0)),
        ],
        out_shape=[
            jax.ShapeDtypeStruct((NP, 1), jnp.float32),
            jax.ShapeDtypeStruct((NP, 1), jnp.float32),
        ],
    )(agg, rin, wr4, wk4, br4)


def _tc5_body(agg4_ref, r4_ref, out_ref):
    s = agg4_ref[0] + agg4_ref[1] + r4_ref[...]
    out_ref[...] = 1.0 / (1.0 + jnp.exp(-s))


@jax.jit
def _tc5(agg4, r4):
    return pl.pallas_call(
        _tc5_body,
        grid=(GRID,),
        in_specs=[
            pl.BlockSpec((2, BLK, 1), lambda i: (0, i, 0)),
            pl.BlockSpec((BLK, 1), lambda i: (i, 0)),
        ],
        out_specs=pl.BlockSpec((BLK, 1), lambda i: (i, 0)),
        out_shape=jax.ShapeDtypeStruct((NP, 1), jnp.float32),
    )(agg4, r4)


def kernel(x, edge_index, edge_weight, Wrel0, brel0, Wroot0, Wrel1, brel1,
           Wroot1, Wrel2, brel2, Wroot2, Wrel3, brel3, Wroot3, Wrel4, brel4,
           Wroot4):
    ne = edge_index.shape[1]
    pad_e = NE - ne
    src = jnp.concatenate(
        [edge_index[0].astype(jnp.int32),
         jnp.full((pad_e,), NP - 1, jnp.int32)])
    dst2d = jnp.concatenate(
        [edge_index[1].astype(jnp.int32),
         jnp.full((pad_e,), NP - 1, jnp.int32)]).reshape(NEROWS, BATCH)
    ew2d = jnp.concatenate(
        [edge_weight, jnp.zeros((pad_e,), jnp.float32)]).reshape(NEROWS, BATCH)
    src2d = src.reshape(NEROWS, BATCH)
    srcoff = jnp.concatenate(
        [src2d, src2d + NP], axis=0)

    xflat = jnp.pad(x[:, 0], (0, NP - N_NODES))

    agg0 = _sc_agg1(xflat, src2d, dst2d, ew2d).reshape(2, NP, 1)
    y1, r1 = _tc1(xflat.reshape(NP, 1), agg0, Wrel0.reshape(1, 64), brel0.reshape(1, 64),
                  Wroot0.reshape(1, 64), Wrel1, Wroot1, brel1.reshape(1, 64))
    a1 = _sc_agg64(y1.reshape(2 * NP, HALF), srcoff, dst2d, ew2d)
    y2, r2 = _tcmid(a1.reshape(2, NP, HALF), r1, Wrel2, Wroot2,
                    brel2.reshape(1, 64))
    a2 = _sc_agg64(y2.reshape(2 * NP, HALF), srcoff, dst2d, ew2d)
    y3, r3 = _tcmid(a2.reshape(2, NP, HALF), r2, Wrel3, Wroot3,
                    brel3.reshape(1, 64))
    a3 = _sc_agg64(y3.reshape(2 * NP, HALF), srcoff, dst2d, ew2d)
    y4, r4 = _tc4(a3.reshape(2, NP, HALF), r3, Wrel4, Wroot4,
                  brel4.reshape(1, 1))
    agg4 = _sc_agg1(y4.reshape(NP), src2d, dst2d, ew2d).reshape(2, NP, 1)
    out2d = _tc5(agg4, r4)
    return out2d[:N_NODES]
